# Initial kernel scaffold; baseline (speedup 1.0000x reference)
import jax
import jax.numpy as jnp
from jax.experimental import pallas as pl


def kernel(x, edge_index, W1, a_src1, a_dst1, b1, W2, a_src2, a_dst2, b2):
    raise NotImplementedError("write your pallas kernel here")


# speedup vs baseline: 32.3823x; 32.3823x over previous
import functools
import jax
import jax.numpy as jnp
from jax import lax
from jax.experimental import pallas as pl
from jax.experimental.pallas import tpu as pltpu
from jax.experimental.pallas import tpu_sc as plsc

N = 10000
F = 128
NC = 2
NS = 16
NW = NC * NS
L = 16
NPAD = 10240
RPT = NPAD // NS
WBC = RPT // 2
ES = 320000 + N
BE = 64
NB = 2 * (-(-ES // (NW * 2 * BE)))
NBR = NB // 2
EPT = NB * BE
NE_PAD = NW * EPT
BLK = 512


def _mm_scores_body(x_ref, w_ref, asrc_ref, adst_ref, h_ref, as_ref, ad_ref):
    h = jnp.dot(x_ref[...], w_ref[...], preferred_element_type=jnp.float32)
    h_ref[...] = h
    as_ref[...] = jnp.sum(h * asrc_ref[...], axis=1, keepdims=True)
    ad_ref[...] = jnp.sum(h * adst_ref[...], axis=1, keepdims=True)


def _tc_mm_scores(x, w, asrc, adst):
    return pl.pallas_call(
        _mm_scores_body,
        grid=(NPAD // BLK,),
        in_specs=[
            pl.BlockSpec((BLK, F), lambda i: (i, 0)),
            pl.BlockSpec((F, F), lambda i: (0, 0)),
            pl.BlockSpec((1, F), lambda i: (0, 0)),
            pl.BlockSpec((1, F), lambda i: (0, 0)),
        ],
        out_specs=[
            pl.BlockSpec((BLK, F), lambda i: (i, 0)),
            pl.BlockSpec((BLK, 1), lambda i: (i, 0)),
            pl.BlockSpec((BLK, 1), lambda i: (i, 0)),
        ],
        out_shape=[
            jax.ShapeDtypeStruct((NPAD, F), jnp.float32),
            jax.ShapeDtypeStruct((NPAD, 1), jnp.float32),
            jax.ShapeDtypeStruct((NPAD, 1), jnp.float32),
        ],
    )(x, w, asrc, adst)


def _elu(x):
    return jnp.where(x > 0, x, jnp.exp(x) - 1.0)


def _norm_mm_body(acc_ref, den_ref, b_ref, w_ref, asrc_ref, adst_ref,
                  h_ref, as_ref, ad_ref):
    a = acc_ref[0] + acc_ref[1]
    d = den_ref[0] + den_ref[1] + 1e-16
    xx = _elu(a / d + b_ref[...])
    h = jnp.dot(xx, w_ref[...], preferred_element_type=jnp.float32)
    h_ref[...] = h
    as_ref[...] = jnp.sum(h * asrc_ref[...], axis=1, keepdims=True)
    ad_ref[...] = jnp.sum(h * adst_ref[...], axis=1, keepdims=True)


def _tc_norm_mm(acc, den, b, w, asrc, adst):
    return pl.pallas_call(
        _norm_mm_body,
        grid=(NPAD // BLK,),
        in_specs=[
            pl.BlockSpec((NC, BLK, F), lambda i: (0, i, 0)),
            pl.BlockSpec((NC, BLK, 1), lambda i: (0, i, 0)),
            pl.BlockSpec((1, F), lambda i: (0, 0)),
            pl.BlockSpec((F, F), lambda i: (0, 0)),
            pl.BlockSpec((1, F), lambda i: (0, 0)),
            pl.BlockSpec((1, F), lambda i: (0, 0)),
        ],
        out_specs=[
            pl.BlockSpec((BLK, F), lambda i: (i, 0)),
            pl.BlockSpec((BLK, 1), lambda i: (i, 0)),
            pl.BlockSpec((BLK, 1), lambda i: (i, 0)),
        ],
        out_shape=[
            jax.ShapeDtypeStruct((NPAD, F), jnp.float32),
            jax.ShapeDtypeStruct((NPAD, 1), jnp.float32),
            jax.ShapeDtypeStruct((NPAD, 1), jnp.float32),
        ],
    )(acc, den, b, w, asrc, adst)


def _final_body(acc_ref, den_ref, b_ref, o_ref):
    a = acc_ref[0] + acc_ref[1]
    d = den_ref[0] + den_ref[1] + 1e-16
    o_ref[...] = _elu(a / d + b_ref[...])


def _tc_final(acc, den, b):
    return pl.pallas_call(
        _final_body,
        grid=(NPAD // BLK,),
        in_specs=[
            pl.BlockSpec((NC, BLK, F), lambda i: (0, i, 0)),
            pl.BlockSpec((NC, BLK, 1), lambda i: (0, i, 0)),
            pl.BlockSpec((1, F), lambda i: (0, 0)),
        ],
        out_specs=pl.BlockSpec((BLK, F), lambda i: (i, 0)),
        out_shape=jax.ShapeDtypeStruct((NPAD, F), jnp.float32),
    )(acc, den, b)


def _sc_body(h_hbm, src_hbm, dst_hbm, as_hbm, ad_hbm,
             acc_hbm, den_hbm,
             src_v, dst_v, rows0_v, rows1_v, asg_v, adg_v, exb_v, dstb_v,
             dtmp_v, acc_sh, den_sh,
             semr0, semr1, sema0, sema1, semb0, semb1):
    c = lax.axis_index("c")
    s = lax.axis_index("s")
    w = c * NS + s

    pltpu.sync_copy(src_hbm.at[w], src_v)
    pltpu.sync_copy(dst_hbm.at[w], dst_v)

    def _zrow(r, carry):
        for j in range(F // L):
            rows0_v[r, pl.ds(j * L, L)] = jnp.zeros((L,), jnp.float32)
        return carry
    lax.fori_loop(0, BE, _zrow, 0)

    def _zden(i, carry):
        dtmp_v[pl.ds(i * L, L)] = jnp.zeros((L,), jnp.float32)
        return carry
    lax.fori_loop(0, RPT // L, _zden, 0)

    base = s * RPT
    for ch in range(RPT // BE):
        pltpu.sync_copy(rows0_v, acc_sh.at[pl.ds(base + ch * BE, BE)])
    pltpu.sync_copy(dtmp_v, den_sh.at[pl.ds(base, RPT)])
    plsc.subcore_barrier()

    rows = (rows0_v, rows1_v)
    semr = (semr0, semr1)
    sema = (sema0, sema1)
    semb = (semb0, semb1)

    def _issue(row, half, buf):
        sidx = src_v.at[row, pl.ds(half * BE, BE)]
        didx = dst_v.at[row, pl.ds(half * BE, BE)]
        pltpu.async_copy(h_hbm.at[sidx], rows[buf], semr[buf])
        pltpu.async_copy(as_hbm.at[sidx], asg_v.at[buf], sema[buf])
        pltpu.async_copy(ad_hbm.at[didx], adg_v.at[buf], semb[buf])

    def _wait(buf):
        pltpu.make_async_copy(h_hbm.at[src_v.at[0, pl.ds(0, BE)]],
                              rows[buf], semr[buf]).wait()
        pltpu.make_async_copy(as_hbm.at[src_v.at[0, pl.ds(0, BE)]],
                              asg_v.at[buf], sema[buf]).wait()
        pltpu.make_async_copy(ad_hbm.at[src_v.at[0, pl.ds(0, BE)]],
                              adg_v.at[buf], semb[buf]).wait()

    def _process(row, half, buf):
        bufr = rows[buf]
        for sub in range(BE // L):
            dstb_v[buf, pl.ds(sub * L, L)] = (
                dst_v[row, pl.ds(half * BE + sub * L, L)])
        exs = []
        for sub in range(BE // L):
            sl = pl.ds(sub * L, L)
            e = asg_v[buf, sl] + adg_v[buf, sl]
            e = jnp.where(e > 0, e, 0.2 * e)
            ex = jnp.exp(e)
            exb_v[buf, sl] = ex
            exs.append(ex)
        pltpu.sync_copy(exb_v.at[buf], den_sh.at[dstb_v.at[buf]], add=True)
        for sub in range(BE // L):
            for k in range(L):
                exk = exs[sub][k]
                r = sub * L + k
                for j in range(F // L):
                    sl = pl.ds(j * L, L)
                    bufr[r, sl] = bufr[r, sl] * exk
        pltpu.sync_copy(bufr, acc_sh.at[dstb_v.at[buf]], add=True)

    _issue(0, 0, 0)

    def _it(i, carry):
        _issue(i, 1, 1)
        _wait(0)
        _process(i, 0, 0)

        @pl.when(i < NBR - 1)
        def _():
            _issue(i + 1, 0, 0)

        _wait(1)
        _process(i, 1, 1)
        return carry

    lax.fori_loop(0, NBR, _it, 0)
    plsc.subcore_barrier()

    for ch in range(RPT // BE):
        r0 = base + ch * BE
        pltpu.sync_copy(acc_sh.at[pl.ds(r0, BE)], rows0_v)
        pltpu.sync_copy(rows0_v, acc_hbm.at[c, pl.ds(r0, BE)])
    pltpu.sync_copy(den_sh.at[pl.ds(base, RPT)], dtmp_v)
    pltpu.sync_copy(dtmp_v, den_hbm.at[c, pl.ds(base, RPT)])


_sc_mesh = plsc.VectorSubcoreMesh(
    core_axis_name="c", subcore_axis_name="s", num_cores=NC, num_subcores=NS)

_sc_aggregate = functools.partial(
    pl.kernel,
    out_type=[
        jax.ShapeDtypeStruct((NC, NPAD, F), jnp.float32),
        jax.ShapeDtypeStruct((NC, NPAD), jnp.float32),
    ],
    mesh=_sc_mesh,
    scratch_types=[
        pltpu.VMEM((NBR, 2 * BE), jnp.int32),
        pltpu.VMEM((NBR, 2 * BE), jnp.int32),
        pltpu.VMEM((BE, F), jnp.float32),
        pltpu.VMEM((BE, F), jnp.float32),
        pltpu.VMEM((2, BE), jnp.float32),
        pltpu.VMEM((2, BE), jnp.float32),
        pltpu.VMEM((2, BE), jnp.float32),
        pltpu.VMEM((2, BE), jnp.int32),
        pltpu.VMEM((RPT,), jnp.float32),
        pltpu.VMEM_SHARED((NPAD, F), jnp.float32),
        pltpu.VMEM_SHARED((NPAD,), jnp.float32),
        pltpu.SemaphoreType.DMA,
        pltpu.SemaphoreType.DMA,
        pltpu.SemaphoreType.DMA,
        pltpu.SemaphoreType.DMA,
        pltpu.SemaphoreType.DMA,
        pltpu.SemaphoreType.DMA,
    ],
)(_sc_body)


@jax.jit
def kernel(x, edge_index, W1, a_src1, a_dst1, b1, W2, a_src2, a_dst2, b2):
    x = x.astype(jnp.float32)
    ei = edge_index.astype(jnp.int32)
    loop = jnp.arange(N, dtype=jnp.int32)
    pad = jnp.full((NE_PAD - ES,), N, dtype=jnp.int32)
    src = jnp.concatenate([ei[0], loop, pad]).reshape(NW, NBR, 2 * BE)
    dst = jnp.concatenate([ei[1], loop, pad]).reshape(NW, NBR, 2 * BE)

    xp = jnp.pad(x, ((0, NPAD - N), (0, 0)))

    h1, as1, ad1 = _tc_mm_scores(
        xp, W1, a_src1.reshape(1, F), a_dst1.reshape(1, F))
    acc1, den1 = _sc_aggregate(
        h1, src, dst, as1.reshape(NPAD), ad1.reshape(NPAD))
    h2, as2, ad2 = _tc_norm_mm(
        acc1, den1.reshape(NC, NPAD, 1), b1.reshape(1, F), W2,
        a_src2.reshape(1, F), a_dst2.reshape(1, F))
    acc2, den2 = _sc_aggregate(
        h2, src, dst, as2.reshape(NPAD), ad2.reshape(NPAD))
    out = _tc_final(acc2, den2.reshape(NC, NPAD, 1), b2.reshape(1, F))
    return out[:N]
